# baseline (speedup 1.0000x reference)
import functools

import jax
import jax.numpy as jnp
from jax import lax
from jax.experimental import pallas as pl
from jax.experimental.pallas import tpu as pltpu
from jax.experimental.pallas import tpu_sc as plsc

_N = 8388608
_T, _C, _H, _W = 20, 2, 480, 640
_NBINS = _T * _C * _H * _W

_NSC = 2
_NTILE = 16
_NPASS = 4
_R = _NBINS // (_NPASS * _NSC)
_DUMP = _NTILE * 128
_RD = _R + _DUMP
_ZB = _RD // _NTILE // 8
_CHUNK = 4096
_NG = _CHUNK // 16
_EPT = _N // _NTILE
_NCHUNK = _EPT // _CHUNK


def _minmax(t2):
    g = t2.shape[0] // 1024

    def body(t_ref, mn_ref, mx_ref):
        i = pl.program_id(0)
        m = jnp.min(t_ref[...])
        M = jnp.max(t_ref[...])

        @pl.when(i == 0)
        def _():
            mn_ref[0, 0] = m
            mx_ref[0, 0] = M

        @pl.when(i > 0)
        def _():
            mn_ref[0, 0] = jnp.minimum(mn_ref[0, 0], m)
            mx_ref[0, 0] = jnp.maximum(mx_ref[0, 0], M)

    return pl.pallas_call(
        body,
        grid=(g,),
        in_specs=[pl.BlockSpec((1024, t2.shape[1]), lambda i: (i, 0))],
        out_specs=[
            pl.BlockSpec((1, 1), lambda i: (0, 0), memory_space=pltpu.SMEM),
            pl.BlockSpec((1, 1), lambda i: (0, 0), memory_space=pltpu.SMEM),
        ],
        out_shape=[
            jax.ShapeDtypeStruct((1, 1), jnp.float32),
            jax.ShapeDtypeStruct((1, 1), jnp.float32),
        ],
    )(t2)


def _flat_index(x2, y2, p2, t2, mn, mx):
    rows, cols = x2.shape
    blk = 512
    g = rows // blk

    def body(mn_ref, mx_ref, x_ref, y_ref, p_ref, t_ref, o_ref):
        tmin = mn_ref[0, 0]
        tmax = mx_ref[0, 0]
        has_range = tmax > tmin
        denom = jnp.where(has_range, tmax - tmin, jnp.float32(1.0))
        t = t_ref[...]
        tn = jnp.where(has_range, (t - tmin) / denom * jnp.float32(_T - 1),
                       jnp.zeros_like(t))
        ti = jnp.clip(jnp.round(tn).astype(jnp.int32), 0, _T - 1)
        xc = jnp.clip(x_ref[...], 0, _W - 1)
        yc = jnp.clip(y_ref[...], 0, _H - 1)
        o_ref[...] = ((ti * _C + p_ref[...]) * _H + yc) * _W + xc

    return pl.pallas_call(
        body,
        grid=(g,),
        in_specs=[
            pl.BlockSpec(memory_space=pltpu.SMEM),
            pl.BlockSpec(memory_space=pltpu.SMEM),
            pl.BlockSpec((blk, cols), lambda i: (i, 0)),
            pl.BlockSpec((blk, cols), lambda i: (i, 0)),
            pl.BlockSpec((blk, cols), lambda i: (i, 0)),
            pl.BlockSpec((blk, cols), lambda i: (i, 0)),
        ],
        out_specs=pl.BlockSpec((blk, cols), lambda i: (i, 0)),
        out_shape=jax.ShapeDtypeStruct((rows, cols), jnp.int32),
    )(mn, mx, x2, y2, p2, t2)


def _sc_histogram(flat_idx, zeros_hbm):
    mesh = plsc.VectorSubcoreMesh(
        core_axis_name="c", subcore_axis_name="s",
        num_cores=_NSC, num_subcores=_NTILE)

    @functools.partial(
        pl.kernel,
        out_type=jax.ShapeDtypeStruct((_NBINS,), jnp.float32),
        mesh=mesh,
        scratch_types=[
            pltpu.VMEM((_CHUNK,), jnp.int32),
            pltpu.VMEM((_CHUNK,), jnp.int32),
            pltpu.VMEM((_CHUNK,), jnp.int32),
            pltpu.VMEM((_CHUNK,), jnp.int32),
            pltpu.VMEM((_CHUNK,), jnp.float32),
            pltpu.VMEM_SHARED((_RD,), jnp.float32),
            pltpu.SemaphoreType.DMA((2,)),
            pltpu.SemaphoreType.DMA((2,)),
        ],
    )
    def hist(idx_hbm, z_hbm, out_hbm, in_v0, in_v1, sc_v0, sc_v1,
             ones_v, bins_sh, in_sem, sc_sem):
        in_v = (in_v0, in_v1)
        sc_v = (sc_v0, sc_v1)
        c = lax.axis_index("c")
        s = lax.axis_index("s")
        lane = lax.iota(jnp.int32, 16)

        def fill_ones(i, carry):
            ones_v[pl.ds(pl.multiple_of(i * 16, 16), 16)] = (
                jnp.full((16,), 1.0, jnp.float32))
            return carry

        lax.fori_loop(0, _CHUNK // 16, fill_ones, 0)

        dumpbase = _R + s * 128
        zoff = s * (_RD // _NTILE)
        woff = s * (_R // _NTILE)
        ebase = s * _EPT

        def start_in(g, b):
            return pltpu.async_copy(
                idx_hbm.at[pl.ds(ebase + g * _CHUNK, _CHUNK)],
                in_v[b], in_sem.at[b])

        dumpvecs = [plsc.bitcast(dumpbase + k * 16 + lane, jnp.uint32)
                    for k in range(8)]

        def redirect(b, lo):
            def grp(i, carry):
                for u in range(8):
                    o = pl.multiple_of(i * 128 + u * 16, 16)
                    v = in_v[b][pl.ds(o, 16)]
                    uu = plsc.bitcast(v - lo, jnp.uint32)
                    m = jnp.minimum(uu, dumpvecs[u])
                    sc_v[b][pl.ds(o, 16)] = plsc.bitcast(m, jnp.int32)
                return carry

            lax.fori_loop(0, _NG // 8, grp, 0)

        def start_scatter(b):
            return pltpu.async_copy(ones_v, bins_sh.at[sc_v[b]],
                                    sc_sem.at[b], add=True)

        def wait_in(g, b):
            pltpu.make_async_copy(
                idx_hbm.at[pl.ds(ebase + g * _CHUNK, _CHUNK)],
                in_v[b], in_sem.at[b]).wait()

        def wait_scatter(b):
            pltpu.make_async_copy(ones_v, bins_sh.at[sc_v[b]],
                                  sc_sem.at[b]).wait()

        def pass_body(g, carry):
            r = _NSC * g + c
            lo = r * _R
            start_in(0, 0)
            start_in(1, 1)
            pltpu.sync_copy(z_hbm.at[pl.ds(zoff, _RD // _NTILE)],
                            bins_sh.at[pl.ds(zoff, _RD // _NTILE)])
            plsc.subcore_barrier()

            def chunk_body(k, carry2):
                for b in range(2):
                    gg = 2 * k + b
                    wait_in(gg, b)

                    @pl.when(gg >= 2)
                    def _():
                        wait_scatter(b)

                    redirect(b, lo)
                    start_scatter(b)

                    @pl.when(gg + 2 < _NCHUNK)
                    def _():
                        start_in(gg + 2, b)
                return carry2

            lax.fori_loop(0, _NCHUNK // 2, chunk_body, 0)
            for b in range(2):
                wait_scatter(b)

            plsc.subcore_barrier()
            pltpu.sync_copy(bins_sh.at[pl.ds(woff, _R // _NTILE)],
                            out_hbm.at[pl.ds(lo + woff, _R // _NTILE)])
            plsc.subcore_barrier()
            return carry

        lax.fori_loop(0, _NPASS, pass_body, 0)

    return hist(flat_idx, zeros_hbm)


def _normalize(counts3):
    def body(c_ref, o_ref):
        v = c_ref[...]
        m = jnp.max(v)
        o_ref[...] = jnp.where(m > 0, v / m, v)

    return pl.pallas_call(
        body,
        grid=(_T,),
        in_specs=[pl.BlockSpec((1, _H, 2 * _W), lambda i: (i, 0, 0))],
        out_specs=pl.BlockSpec((1, _H, 2 * _W), lambda i: (i, 0, 0)),
        out_shape=jax.ShapeDtypeStruct(counts3.shape, jnp.float32),
    )(counts3)


def kernel(x, y, p, t):
    t2 = t.reshape(8192, 1024)
    mn, mx = _minmax(t2)
    flat = _flat_index(x.reshape(8192, 1024), y.reshape(8192, 1024),
                       p.reshape(8192, 1024), t2, mn, mx)
    counts = _sc_histogram(flat.reshape(_N), jnp.zeros((_RD,), jnp.float32))
    voxel = _normalize(counts.reshape(_T, _H, 2 * _W))
    return voxel.reshape(_T, _C, _H, _W)


# speedup vs baseline: 1.1030x; 1.1030x over previous
import functools

import jax
import jax.numpy as jnp
from jax import lax
from jax.experimental import pallas as pl
from jax.experimental.pallas import tpu as pltpu
from jax.experimental.pallas import tpu_sc as plsc

_N = 8388608
_T, _C, _H, _W = 20, 2, 480, 640
_NBINS = _T * _C * _H * _W

_NSC = 2
_NTILE = 16
_NW = _NSC * _NTILE
_NPASS = 4
_NREG = _NPASS * _NSC
_R = _NBINS // _NREG
_DUMP = 2048
_RD = _R + _DUMP
_LOCBITS = 21

_EP = _N // _NW
_PCHUNK = 4096
_PNCH = _EP // _PCHUNK
_BLK = 2048
_RING = 8192
_SEG = _EP + _BLK


def _minmax(t2):
    g = t2.shape[0] // 1024

    def body(t_ref, mn_ref, mx_ref):
        i = pl.program_id(0)
        m = jnp.min(t_ref[...])
        M = jnp.max(t_ref[...])

        @pl.when(i == 0)
        def _():
            mn_ref[0, 0] = m
            mx_ref[0, 0] = M

        @pl.when(i > 0)
        def _():
            mn_ref[0, 0] = jnp.minimum(mn_ref[0, 0], m)
            mx_ref[0, 0] = jnp.maximum(mx_ref[0, 0], M)

    return pl.pallas_call(
        body,
        grid=(g,),
        in_specs=[pl.BlockSpec((1024, t2.shape[1]), lambda i: (i, 0))],
        out_specs=[
            pl.BlockSpec((1, 1), lambda i: (0, 0), memory_space=pltpu.SMEM),
            pl.BlockSpec((1, 1), lambda i: (0, 0), memory_space=pltpu.SMEM),
        ],
        out_shape=[
            jax.ShapeDtypeStruct((1, 1), jnp.float32),
            jax.ShapeDtypeStruct((1, 1), jnp.float32),
        ],
    )(t2)


def _packed_index(x2, y2, p2, t2, mn, mx):
    rows, cols = x2.shape
    blk = 512
    g = rows // blk

    def body(mn_ref, mx_ref, x_ref, y_ref, p_ref, t_ref, o_ref):
        tmin = mn_ref[0, 0]
        tmax = mx_ref[0, 0]
        has_range = tmax > tmin
        denom = jnp.where(has_range, tmax - tmin, jnp.float32(1.0))
        t = t_ref[...]
        tn = jnp.where(has_range, (t - tmin) / denom * jnp.float32(_T - 1),
                       jnp.zeros_like(t))
        ti = jnp.clip(jnp.round(tn).astype(jnp.int32), 0, _T - 1)
        xc = jnp.clip(x_ref[...], 0, _W - 1)
        yc = jnp.clip(y_ref[...], 0, _H - 1)
        flat = ((ti * _C + p_ref[...]) * _H + yc) * _W + xc
        r = jnp.zeros_like(flat)
        for k in range(1, _NREG):
            r = r + (flat >= k * _R).astype(jnp.int32)
        local = flat - r * _R
        o_ref[...] = (r << _LOCBITS) | local

    return pl.pallas_call(
        body,
        grid=(g,),
        in_specs=[
            pl.BlockSpec(memory_space=pltpu.SMEM),
            pl.BlockSpec(memory_space=pltpu.SMEM),
            pl.BlockSpec((blk, cols), lambda i: (i, 0)),
            pl.BlockSpec((blk, cols), lambda i: (i, 0)),
            pl.BlockSpec((blk, cols), lambda i: (i, 0)),
            pl.BlockSpec((blk, cols), lambda i: (i, 0)),
        ],
        out_specs=pl.BlockSpec((blk, cols), lambda i: (i, 0)),
        out_shape=jax.ShapeDtypeStruct((rows, cols), jnp.int32),
    )(mn, mx, x2, y2, p2, t2)


def _sc_partition(packed):
    mesh = plsc.VectorSubcoreMesh(
        core_axis_name="c", subcore_axis_name="s",
        num_cores=_NSC, num_subcores=_NTILE)

    scratch = (
        [pltpu.VMEM((_PCHUNK,), jnp.int32) for _ in range(2)]
        + [pltpu.VMEM((_RING,), jnp.int32) for _ in range(_NREG)]
        + [pltpu.VMEM((16,), jnp.int32)]
        + [pltpu.SemaphoreType.DMA((2,)),
           pltpu.SemaphoreType.DMA((_NREG,))]
    )

    @functools.partial(
        pl.kernel,
        out_type=[
            jax.ShapeDtypeStruct((_NW * _NREG * _SEG,), jnp.int32),
            jax.ShapeDtypeStruct((_NW * 16,), jnp.int32),
        ],
        mesh=mesh,
        scratch_types=scratch,
        compiler_params=pltpu.CompilerParams(needs_layout_passes=False),
    )
    def part(pk_hbm, seg_hbm, cnt_hbm, in_v0, in_v1,
             b0, b1, b2, b3, b4, b5, b6, b7, cnt_v, in_sem, fl_sem):
        in_v = (in_v0, in_v1)
        rings = (b0, b1, b2, b3, b4, b5, b6, b7)
        c = lax.axis_index("c")
        s = lax.axis_index("s")
        w = s * _NSC + c
        lane = lax.iota(jnp.int32, 16)
        ebase = w * _EP

        def start_in(g, b):
            pltpu.async_copy(pk_hbm.at[pl.ds(ebase + g * _PCHUNK, _PCHUNK)],
                             in_v[b], in_sem.at[b])

        def wait_in(g, b):
            pltpu.make_async_copy(
                pk_hbm.at[pl.ds(ebase + g * _PCHUNK, _PCHUNK)],
                in_v[b], in_sem.at[b]).wait()

        def flush_desc(rr, off, blkno):
            sbase = (w * _NREG + rr) * _SEG
            return pltpu.make_async_copy(
                rings[rr].at[pl.ds(pl.multiple_of(off, _BLK), _BLK)],
                seg_hbm.at[pl.ds(pl.multiple_of(sbase + blkno * _BLK, _BLK),
                                 _BLK)],
                fl_sem.at[rr])

        def group(b, i, bases):
            o = pl.multiple_of(i * 16, 16)
            v = in_v[b][pl.ds(o, 16)]
            rr_v = lax.shift_right_logical(v, _LOCBITS)
            local = lax.bitwise_and(v, (1 << _LOCBITS) - 1)
            new = []
            for rr in range(_NREG):
                m = rr_v == rr
                mi = m.astype(jnp.int32)
                scan = plsc.cumsum(mi)
                pos = lax.bitwise_and(bases[rr] + (scan - 1), _RING - 1)
                plsc.store_scatter(rings[rr], [pos], local, mask=m)
                new.append(bases[rr] + jnp.max(scan))
            return tuple(new)

        def flush_two(bases, flushed, blks):
            nf, nb = [], []
            for rr in range(_NREG):
                fl = flushed[rr]
                bl = blks[rr]
                for _ in range(2):
                    do = (bases[rr] - fl) >= _BLK

                    @pl.when(do & (bl > 0))
                    def _():
                        flush_desc(rr, 0, 0).wait()

                    @pl.when(do)
                    def _():
                        flush_desc(rr, lax.bitwise_and(fl, _RING - 1),
                                   bl).start()

                    fl = jnp.where(do, fl + _BLK, fl)
                    bl = jnp.where(do, bl + 1, bl)
                nf.append(fl)
                nb.append(bl)
            return tuple(nf), tuple(nb)

        def chunk(k2, state):
            bases, flushed, blks = state
            for b in range(2):
                g = 2 * k2 + b
                wait_in(g, b)

                for half in range(2):
                    def gbody(i, bs):
                        return group(b, i, bs)

                    bases = lax.fori_loop(half * (_BLK // 16),
                                          (half + 1) * (_BLK // 16),
                                          gbody, bases)
                    flushed, blks = flush_two(bases, flushed, blks)

                @pl.when(g + 2 < _PNCH)
                def _():
                    start_in(g + 2, b)
            return bases, flushed, blks

        start_in(0, 0)
        start_in(1, 1)
        z = jnp.int32(0)
        state = ((z,) * _NREG, (z,) * _NREG, (z,) * _NREG)
        bases, flushed, blks = lax.fori_loop(0, _PNCH // 2, chunk, state)

        cvec = jnp.zeros((16,), jnp.int32)
        for rr in range(_NREG):
            dump = _R + lax.bitwise_and(w * 64 + rr * 16 + lane, _DUMP - 1)
            base = bases[rr]

            def padb(i, carry):
                pos = lax.bitwise_and(base + i * 16 + lane, _RING - 1)
                plsc.store_scatter(rings[rr], [pos], dump)
                return carry

            lax.fori_loop(0, _BLK // 16, padb, 0)
            gap = lax.bitwise_and(-base, _BLK - 1)
            padded = base + gap
            fl = flushed[rr]
            bl = blks[rr]
            for _ in range(2):
                do = (padded - fl) >= _BLK

                @pl.when(do & (bl > 0))
                def _():
                    flush_desc(rr, 0, 0).wait()

                @pl.when(do)
                def _():
                    flush_desc(rr, lax.bitwise_and(fl, _RING - 1),
                               bl).start()

                fl = jnp.where(do, fl + _BLK, fl)
                bl = jnp.where(do, bl + 1, bl)

            @pl.when(bl > 0)
            def _():
                flush_desc(rr, 0, 0).wait()

            cvec = jnp.where(lane == rr, bl, cvec)

        cnt_v[...] = cvec
        pltpu.sync_copy(cnt_v, cnt_hbm.at[pl.ds(w * 16, 16)])

    return part(packed)


def _sc_scatter(seg_hbm, cnt_hbm, zeros_hbm):
    mesh = plsc.VectorSubcoreMesh(
        core_axis_name="c", subcore_axis_name="s",
        num_cores=_NSC, num_subcores=_NTILE)

    scratch = (
        [pltpu.VMEM((_BLK,), jnp.int32) for _ in range(4)]
        + [pltpu.VMEM((_BLK,), jnp.float32),
           pltpu.VMEM((16,), jnp.int32),
           pltpu.VMEM((16,), jnp.int32),
           pltpu.VMEM_SHARED((_RD,), jnp.float32)]
        + [pltpu.SemaphoreType.DMA((4,)),
           pltpu.SemaphoreType.DMA((4,))]
    )

    @functools.partial(
        pl.kernel,
        out_type=jax.ShapeDtypeStruct((_NBINS,), jnp.float32),
        mesh=mesh,
        scratch_types=scratch,
        compiler_params=pltpu.CompilerParams(needs_layout_passes=False),
    )
    def scat(seg, cnt, z_hbm, out_hbm, i0, i1, i2, i3, ones_v,
             ca_v, cb_v, bins_sh, in_sem, sc_sem):
        in_v = (i0, i1, i2, i3)
        c = lax.axis_index("c")
        s = lax.axis_index("s")
        lane = lax.iota(jnp.int32, 16)

        def fill_ones(i, carry):
            ones_v[pl.ds(pl.multiple_of(i * 16, 16), 16)] = (
                jnp.full((16,), 1.0, jnp.float32))
            return carry

        lax.fori_loop(0, _BLK // 16, fill_ones, 0)

        zoff = s * (_RD // _NTILE)
        woff = s * (_R // _NTILE)
        wa = 2 * s
        wb = 2 * s + 1

        def pass_body(k, carry):
            rr = _NSC * k + c
            lo = rr * _R
            pltpu.sync_copy(cnt.at[pl.ds(wa * 16, 16)], ca_v)
            pltpu.sync_copy(cnt.at[pl.ds(wb * 16, 16)], cb_v)
            na = jnp.max(jnp.where(lane == rr, ca_v[pl.ds(0, 16)], 0))
            nb = jnp.max(jnp.where(lane == rr, cb_v[pl.ds(0, 16)], 0))
            ntot = na + nb
            sba = (wa * _NREG + rr) * _SEG
            sbb = (wb * _NREG + rr) * _SEG

            def blk_src(i):
                off = jnp.where(i < na, sba + i * _BLK,
                                sbb + (i - na) * _BLK)
                return pl.ds(pl.multiple_of(off, 8), _BLK)

            def start_in(i, b):
                pltpu.async_copy(seg.at[blk_src(i)], in_v[b], in_sem.at[b])

            def wait_in(i, b):
                pltpu.make_async_copy(seg.at[blk_src(i)], in_v[b],
                                      in_sem.at[b]).wait()

            def start_scatter(b):
                pltpu.async_copy(ones_v, bins_sh.at[in_v[b]],
                                 sc_sem.at[b], add=True)

            def wait_scatter(b):
                pltpu.make_async_copy(ones_v, bins_sh.at[in_v[b]],
                                      sc_sem.at[b]).wait()

            pltpu.sync_copy(z_hbm.at[pl.ds(zoff, _RD // _NTILE)],
                            bins_sh.at[pl.ds(zoff, _RD // _NTILE)])

            @pl.when(ntot > 0)
            def _():
                start_in(0, 0)

            plsc.subcore_barrier()

            def consume4(k4, carry2):
                for bb in range(4):
                    i = 4 * k4 + bb
                    bn = (bb + 1) % 4

                    @pl.when(i < ntot)
                    def _():
                        wait_in(i, bb)

                        @pl.when(i >= 3)
                        def _():
                            wait_scatter(bn)

                        @pl.when(i + 1 < ntot)
                        def _():
                            start_in(i + 1, bn)

                        start_scatter(bb)
                return carry2

            lax.fori_loop(0, lax.div(ntot + 3, 4), consume4, 0)

            for bb in range(4):
                cond = jnp.bool_(False)
                for j in range(1, 4):
                    cond = cond | ((ntot >= j) &
                                   (lax.rem(ntot - j, 4) == bb))

                @pl.when(cond)
                def _():
                    wait_scatter(bb)

            plsc.subcore_barrier()
            pltpu.sync_copy(bins_sh.at[pl.ds(woff, _R // _NTILE)],
                            out_hbm.at[pl.ds(lo + woff, _R // _NTILE)])
            plsc.subcore_barrier()
            return carry

        lax.fori_loop(0, _NPASS, pass_body, 0)

    return scat(seg_hbm, cnt_hbm, zeros_hbm)


def _normalize(counts3):
    def body(c_ref, o_ref):
        v = c_ref[...]
        m = jnp.max(v)
        o_ref[...] = jnp.where(m > 0, v / m, v)

    return pl.pallas_call(
        body,
        grid=(_T,),
        in_specs=[pl.BlockSpec((1, _H, 2 * _W), lambda i: (i, 0, 0))],
        out_specs=pl.BlockSpec((1, _H, 2 * _W), lambda i: (i, 0, 0)),
        out_shape=jax.ShapeDtypeStruct(counts3.shape, jnp.float32),
    )(counts3)


def kernel(x, y, p, t):
    t2 = t.reshape(8192, 1024)
    mn, mx = _minmax(t2)
    packed = _packed_index(x.reshape(8192, 1024), y.reshape(8192, 1024),
                           p.reshape(8192, 1024), t2, mn, mx)
    seg, cnt = _sc_partition(packed.reshape(_N))
    counts = _sc_scatter(seg, cnt, jnp.zeros((_RD,), jnp.float32))
    voxel = _normalize(counts.reshape(_T, _H, 2 * _W))
    return voxel.reshape(_T, _C, _H, _W)


# speedup vs baseline: 1.1779x; 1.0679x over previous
import functools

import jax
import jax.numpy as jnp
from jax import lax
from jax.experimental import pallas as pl
from jax.experimental.pallas import tpu as pltpu
from jax.experimental.pallas import tpu_sc as plsc

_N = 8388608
_T, _C, _H, _W = 20, 2, 480, 640
_NBINS = _T * _C * _H * _W

_NSC = 2
_NTILE = 16
_NW = _NSC * _NTILE
_NPASS = 4
_NREG = _NPASS * _NSC
_R = _NBINS // _NREG
_DUMP = 2048
_RD = _R + _DUMP
_LOCBITS = 21

_EP = _N // _NW
_PCHUNK = 4096
_PNCH = _EP // _PCHUNK
_BLK = 2048
_RING = 8192
_SEG = _EP + _BLK


def _minmax(t2):
    g = t2.shape[0] // 1024

    def body(t_ref, mn_ref, mx_ref):
        i = pl.program_id(0)
        m = jnp.min(t_ref[...])
        M = jnp.max(t_ref[...])

        @pl.when(i == 0)
        def _():
            mn_ref[0, 0] = m
            mx_ref[0, 0] = M

        @pl.when(i > 0)
        def _():
            mn_ref[0, 0] = jnp.minimum(mn_ref[0, 0], m)
            mx_ref[0, 0] = jnp.maximum(mx_ref[0, 0], M)

    return pl.pallas_call(
        body,
        grid=(g,),
        in_specs=[pl.BlockSpec((1024, t2.shape[1]), lambda i: (i, 0))],
        out_specs=[
            pl.BlockSpec((1, 1), lambda i: (0, 0), memory_space=pltpu.SMEM),
            pl.BlockSpec((1, 1), lambda i: (0, 0), memory_space=pltpu.SMEM),
        ],
        out_shape=[
            jax.ShapeDtypeStruct((1, 1), jnp.float32),
            jax.ShapeDtypeStruct((1, 1), jnp.float32),
        ],
    )(t2)


def _packed_index(x2, y2, p2, t2, mn, mx):
    rows, cols = x2.shape
    blk = 512
    g = rows // blk

    def body(mn_ref, mx_ref, x_ref, y_ref, p_ref, t_ref, o_ref):
        tmin = mn_ref[0, 0]
        tmax = mx_ref[0, 0]
        has_range = tmax > tmin
        denom = jnp.where(has_range, tmax - tmin, jnp.float32(1.0))
        t = t_ref[...]
        tn = jnp.where(has_range, (t - tmin) / denom * jnp.float32(_T - 1),
                       jnp.zeros_like(t))
        ti = jnp.clip(jnp.round(tn).astype(jnp.int32), 0, _T - 1)
        xc = jnp.clip(x_ref[...], 0, _W - 1)
        yc = jnp.clip(y_ref[...], 0, _H - 1)
        flat = ((ti * _C + p_ref[...]) * _H + yc) * _W + xc
        r = jnp.zeros_like(flat)
        for k in range(1, _NREG):
            r = r + (flat >= k * _R).astype(jnp.int32)
        local = flat - r * _R
        o_ref[...] = (r << _LOCBITS) | local

    return pl.pallas_call(
        body,
        grid=(g,),
        in_specs=[
            pl.BlockSpec(memory_space=pltpu.SMEM),
            pl.BlockSpec(memory_space=pltpu.SMEM),
            pl.BlockSpec((blk, cols), lambda i: (i, 0)),
            pl.BlockSpec((blk, cols), lambda i: (i, 0)),
            pl.BlockSpec((blk, cols), lambda i: (i, 0)),
            pl.BlockSpec((blk, cols), lambda i: (i, 0)),
        ],
        out_specs=pl.BlockSpec((blk, cols), lambda i: (i, 0)),
        out_shape=jax.ShapeDtypeStruct((rows, cols), jnp.int32),
    )(mn, mx, x2, y2, p2, t2)


def _sc_partition(packed):
    mesh = plsc.VectorSubcoreMesh(
        core_axis_name="c", subcore_axis_name="s",
        num_cores=_NSC, num_subcores=_NTILE)

    scratch = (
        [pltpu.VMEM((_PCHUNK,), jnp.int32) for _ in range(2)]
        + [pltpu.VMEM((_RING,), jnp.int32) for _ in range(_NREG)]
        + [pltpu.VMEM((16,), jnp.int32)]
        + [pltpu.SemaphoreType.DMA((2,)),
           pltpu.SemaphoreType.DMA((_NREG,))]
    )

    @functools.partial(
        pl.kernel,
        out_type=[
            jax.ShapeDtypeStruct((_NW * _NREG * _SEG,), jnp.int32),
            jax.ShapeDtypeStruct((_NW * 16,), jnp.int32),
        ],
        mesh=mesh,
        scratch_types=scratch,
        compiler_params=pltpu.CompilerParams(needs_layout_passes=False),
    )
    def part(pk_hbm, seg_hbm, cnt_hbm, in_v0, in_v1,
             b0, b1, b2, b3, b4, b5, b6, b7, cnt_v, in_sem, fl_sem):
        in_v = (in_v0, in_v1)
        rings = (b0, b1, b2, b3, b4, b5, b6, b7)
        c = lax.axis_index("c")
        s = lax.axis_index("s")
        w = s * _NSC + c
        lane = lax.iota(jnp.int32, 16)
        ebase = w * _EP

        def start_in(g, b):
            pltpu.async_copy(pk_hbm.at[pl.ds(ebase + g * _PCHUNK, _PCHUNK)],
                             in_v[b], in_sem.at[b])

        def wait_in(g, b):
            pltpu.make_async_copy(
                pk_hbm.at[pl.ds(ebase + g * _PCHUNK, _PCHUNK)],
                in_v[b], in_sem.at[b]).wait()

        def flush_desc(rr, off, blkno):
            sbase = (w * _NREG + rr) * _SEG
            return pltpu.make_async_copy(
                rings[rr].at[pl.ds(pl.multiple_of(off, _BLK), _BLK)],
                seg_hbm.at[pl.ds(pl.multiple_of(sbase + blkno * _BLK, _BLK),
                                 _BLK)],
                fl_sem.at[rr])

        def group(b, i, bases):
            o = pl.multiple_of(i * 16, 16)
            v = in_v[b][pl.ds(o, 16)]
            rr_v = lax.shift_right_logical(v, _LOCBITS)
            local = lax.bitwise_and(v, (1 << _LOCBITS) - 1)
            new = []
            for rr in range(_NREG):
                m = rr_v == rr
                mi = m.astype(jnp.int32)
                scan = plsc.cumsum(mi)
                pos = lax.bitwise_and(bases[rr] + (scan - 1), _RING - 1)
                plsc.store_scatter(rings[rr], [pos], local, mask=m)
                tot = plsc.all_reduce_population_count(m)
                new.append(bases[rr] + tot)
            return tuple(new)

        def flush_two(bases, flushed, blks):
            nf, nb = [], []
            for rr in range(_NREG):
                fl = flushed[rr]
                bl = blks[rr]
                bs = jnp.max(bases[rr])
                for _ in range(2):
                    do = (bs - fl) >= _BLK

                    @pl.when(do & (bl > 0))
                    def _():
                        flush_desc(rr, 0, 0).wait()

                    @pl.when(do)
                    def _():
                        flush_desc(rr, lax.bitwise_and(fl, _RING - 1),
                                   bl).start()

                    fl = jnp.where(do, fl + _BLK, fl)
                    bl = jnp.where(do, bl + 1, bl)
                nf.append(fl)
                nb.append(bl)
            return tuple(nf), tuple(nb)

        def chunk(k2, state):
            bases, flushed, blks = state
            for b in range(2):
                g = 2 * k2 + b
                wait_in(g, b)

                for half in range(2):
                    def gbody(i, bs):
                        return group(b, i, bs)

                    bases = lax.fori_loop(half * (_BLK // 16),
                                          (half + 1) * (_BLK // 16),
                                          gbody, bases)
                    flushed, blks = flush_two(bases, flushed, blks)

                @pl.when(g + 2 < _PNCH)
                def _():
                    start_in(g + 2, b)
            return bases, flushed, blks

        start_in(0, 0)
        start_in(1, 1)
        zv = jnp.zeros((16,), jnp.int32)
        z = jnp.int32(0)
        state = ((zv,) * _NREG, (z,) * _NREG, (z,) * _NREG)
        bases, flushed, blks = lax.fori_loop(0, _PNCH // 2, chunk, state)

        cvec = jnp.zeros((16,), jnp.int32)
        for rr in range(_NREG):
            dump = _R + lax.bitwise_and(w * 64 + rr * 16 + lane, _DUMP - 1)
            base = jnp.max(bases[rr])

            def padb(i, carry):
                pos = lax.bitwise_and(base + i * 16 + lane, _RING - 1)
                plsc.store_scatter(rings[rr], [pos], dump)
                return carry

            lax.fori_loop(0, _BLK // 16, padb, 0)
            gap = lax.bitwise_and(-base, _BLK - 1)
            padded = base + gap
            fl = flushed[rr]
            bl = blks[rr]
            for _ in range(2):
                do = (padded - fl) >= _BLK

                @pl.when(do & (bl > 0))
                def _():
                    flush_desc(rr, 0, 0).wait()

                @pl.when(do)
                def _():
                    flush_desc(rr, lax.bitwise_and(fl, _RING - 1),
                               bl).start()

                fl = jnp.where(do, fl + _BLK, fl)
                bl = jnp.where(do, bl + 1, bl)

            @pl.when(bl > 0)
            def _():
                flush_desc(rr, 0, 0).wait()

            cvec = jnp.where(lane == rr, bl, cvec)

        cnt_v[...] = cvec
        pltpu.sync_copy(cnt_v, cnt_hbm.at[pl.ds(w * 16, 16)])

    return part(packed)


def _sc_scatter(seg_hbm, cnt_hbm, zeros_hbm):
    mesh = plsc.VectorSubcoreMesh(
        core_axis_name="c", subcore_axis_name="s",
        num_cores=_NSC, num_subcores=_NTILE)

    scratch = (
        [pltpu.VMEM((_BLK,), jnp.int32) for _ in range(4)]
        + [pltpu.VMEM((_BLK,), jnp.float32),
           pltpu.VMEM((16,), jnp.int32),
           pltpu.VMEM((16,), jnp.int32),
           pltpu.VMEM_SHARED((_RD,), jnp.float32)]
        + [pltpu.SemaphoreType.DMA((4,)),
           pltpu.SemaphoreType.DMA((4,))]
    )

    @functools.partial(
        pl.kernel,
        out_type=jax.ShapeDtypeStruct((_NBINS,), jnp.float32),
        mesh=mesh,
        scratch_types=scratch,
        compiler_params=pltpu.CompilerParams(needs_layout_passes=False),
    )
    def scat(seg, cnt, z_hbm, out_hbm, i0, i1, i2, i3, ones_v,
             ca_v, cb_v, bins_sh, in_sem, sc_sem):
        in_v = (i0, i1, i2, i3)
        c = lax.axis_index("c")
        s = lax.axis_index("s")
        lane = lax.iota(jnp.int32, 16)

        def fill_ones(i, carry):
            ones_v[pl.ds(pl.multiple_of(i * 16, 16), 16)] = (
                jnp.full((16,), 1.0, jnp.float32))
            return carry

        lax.fori_loop(0, _BLK // 16, fill_ones, 0)

        zoff = s * (_RD // _NTILE)
        woff = s * (_R // _NTILE)
        wa = 2 * s
        wb = 2 * s + 1

        def pass_body(k, carry):
            rr = _NSC * k + c
            lo = rr * _R
            pltpu.sync_copy(cnt.at[pl.ds(wa * 16, 16)], ca_v)
            pltpu.sync_copy(cnt.at[pl.ds(wb * 16, 16)], cb_v)
            na = jnp.max(jnp.where(lane == rr, ca_v[pl.ds(0, 16)], 0))
            nb = jnp.max(jnp.where(lane == rr, cb_v[pl.ds(0, 16)], 0))
            ntot = na + nb
            sba = (wa * _NREG + rr) * _SEG
            sbb = (wb * _NREG + rr) * _SEG

            def blk_src(i):
                off = jnp.where(i < na, sba + i * _BLK,
                                sbb + (i - na) * _BLK)
                return pl.ds(pl.multiple_of(off, 8), _BLK)

            def start_in(i, b):
                pltpu.async_copy(seg.at[blk_src(i)], in_v[b], in_sem.at[b])

            def wait_in(i, b):
                pltpu.make_async_copy(seg.at[blk_src(i)], in_v[b],
                                      in_sem.at[b]).wait()

            def start_scatter(b):
                pltpu.async_copy(ones_v, bins_sh.at[in_v[b]],
                                 sc_sem.at[b], add=True)

            def wait_scatter(b):
                pltpu.make_async_copy(ones_v, bins_sh.at[in_v[b]],
                                      sc_sem.at[b]).wait()

            pltpu.sync_copy(z_hbm.at[pl.ds(zoff, _RD // _NTILE)],
                            bins_sh.at[pl.ds(zoff, _RD // _NTILE)])

            @pl.when(ntot > 0)
            def _():
                start_in(0, 0)

            plsc.subcore_barrier()

            def consume4(k4, carry2):
                for bb in range(4):
                    i = 4 * k4 + bb
                    bn = (bb + 1) % 4

                    @pl.when(i < ntot)
                    def _():
                        wait_in(i, bb)

                        @pl.when(i >= 3)
                        def _():
                            wait_scatter(bn)

                        @pl.when(i + 1 < ntot)
                        def _():
                            start_in(i + 1, bn)

                        start_scatter(bb)
                return carry2

            lax.fori_loop(0, lax.div(ntot + 3, 4), consume4, 0)

            for bb in range(4):
                cond = jnp.bool_(False)
                for j in range(1, 4):
                    cond = cond | ((ntot >= j) &
                                   (lax.rem(ntot - j, 4) == bb))

                @pl.when(cond)
                def _():
                    wait_scatter(bb)

            plsc.subcore_barrier()
            pltpu.sync_copy(bins_sh.at[pl.ds(woff, _R // _NTILE)],
                            out_hbm.at[pl.ds(lo + woff, _R // _NTILE)])
            plsc.subcore_barrier()
            return carry

        lax.fori_loop(0, _NPASS, pass_body, 0)

    return scat(seg_hbm, cnt_hbm, zeros_hbm)


def _normalize(counts3):
    def body(c_ref, o_ref):
        v = c_ref[...]
        m = jnp.max(v)
        o_ref[...] = jnp.where(m > 0, v / m, v)

    return pl.pallas_call(
        body,
        grid=(_T,),
        in_specs=[pl.BlockSpec((1, _H, 2 * _W), lambda i: (i, 0, 0))],
        out_specs=pl.BlockSpec((1, _H, 2 * _W), lambda i: (i, 0, 0)),
        out_shape=jax.ShapeDtypeStruct(counts3.shape, jnp.float32),
    )(counts3)


def kernel(x, y, p, t):
    t2 = t.reshape(8192, 1024)
    mn, mx = _minmax(t2)
    packed = _packed_index(x.reshape(8192, 1024), y.reshape(8192, 1024),
                           p.reshape(8192, 1024), t2, mn, mx)
    seg, cnt = _sc_partition(packed.reshape(_N))
    counts = _sc_scatter(seg, cnt, jnp.zeros((_RD,), jnp.float32))
    voxel = _normalize(counts.reshape(_T, _H, 2 * _W))
    return voxel.reshape(_T, _C, _H, _W)


# speedup vs baseline: 1.2051x; 1.0231x over previous
import functools

import jax
import jax.numpy as jnp
from jax import lax
from jax.experimental import pallas as pl
from jax.experimental.pallas import tpu as pltpu
from jax.experimental.pallas import tpu_sc as plsc

_N = 8388608
_T, _C, _H, _W = 20, 2, 480, 640
_NBINS = _T * _C * _H * _W

_NSC = 2
_NTILE = 16
_NW = _NSC * _NTILE
_NPASS = 4
_NREG = _NPASS * _NSC
_R = _NBINS // _NREG
_DUMP = 2048
_RD = _R + _DUMP
_LOCBITS = 21

_EP = _N // _NW
_PCHUNK = 4096
_PNCH = _EP // _PCHUNK
_BLK = 2048
_RING = 8192
_SEG = _EP + _BLK


def _minmax(t2):
    g = t2.shape[0] // 1024

    def body(t_ref, mn_ref, mx_ref):
        i = pl.program_id(0)
        m = jnp.min(t_ref[...])
        M = jnp.max(t_ref[...])

        @pl.when(i == 0)
        def _():
            mn_ref[0, 0] = m
            mx_ref[0, 0] = M

        @pl.when(i > 0)
        def _():
            mn_ref[0, 0] = jnp.minimum(mn_ref[0, 0], m)
            mx_ref[0, 0] = jnp.maximum(mx_ref[0, 0], M)

    return pl.pallas_call(
        body,
        grid=(g,),
        in_specs=[pl.BlockSpec((1024, t2.shape[1]), lambda i: (i, 0))],
        out_specs=[
            pl.BlockSpec((1, 1), lambda i: (0, 0), memory_space=pltpu.SMEM),
            pl.BlockSpec((1, 1), lambda i: (0, 0), memory_space=pltpu.SMEM),
        ],
        out_shape=[
            jax.ShapeDtypeStruct((1, 1), jnp.float32),
            jax.ShapeDtypeStruct((1, 1), jnp.float32),
        ],
    )(t2)


def _packed_index(x2, y2, p2, t2, mn, mx):
    rows, cols = x2.shape
    blk = 512
    g = rows // blk

    def body(mn_ref, mx_ref, x_ref, y_ref, p_ref, t_ref, o_ref):
        tmin = mn_ref[0, 0]
        tmax = mx_ref[0, 0]
        has_range = tmax > tmin
        denom = jnp.where(has_range, tmax - tmin, jnp.float32(1.0))
        t = t_ref[...]
        tn = jnp.where(has_range, (t - tmin) / denom * jnp.float32(_T - 1),
                       jnp.zeros_like(t))
        ti = jnp.clip(jnp.round(tn).astype(jnp.int32), 0, _T - 1)
        xc = jnp.clip(x_ref[...], 0, _W - 1)
        yc = jnp.clip(y_ref[...], 0, _H - 1)
        flat = ((ti * _C + p_ref[...]) * _H + yc) * _W + xc
        r = jnp.zeros_like(flat)
        for k in range(1, _NREG):
            r = r + (flat >= k * _R).astype(jnp.int32)
        local = flat - r * _R
        o_ref[...] = (r << _LOCBITS) | local

    return pl.pallas_call(
        body,
        grid=(g,),
        in_specs=[
            pl.BlockSpec(memory_space=pltpu.SMEM),
            pl.BlockSpec(memory_space=pltpu.SMEM),
            pl.BlockSpec((blk, cols), lambda i: (i, 0)),
            pl.BlockSpec((blk, cols), lambda i: (i, 0)),
            pl.BlockSpec((blk, cols), lambda i: (i, 0)),
            pl.BlockSpec((blk, cols), lambda i: (i, 0)),
        ],
        out_specs=pl.BlockSpec((blk, cols), lambda i: (i, 0)),
        out_shape=jax.ShapeDtypeStruct((rows, cols), jnp.int32),
    )(mn, mx, x2, y2, p2, t2)


def _sc_partition(packed):
    mesh = plsc.VectorSubcoreMesh(
        core_axis_name="c", subcore_axis_name="s",
        num_cores=_NSC, num_subcores=_NTILE)

    scratch = (
        [pltpu.VMEM((4, 1024), jnp.int32) for _ in range(2)]
        + [pltpu.VMEM((_RING,), jnp.int32) for _ in range(_NREG)]
        + [pltpu.VMEM((16,), jnp.int32)]
        + [pltpu.SemaphoreType.DMA((2,)),
           pltpu.SemaphoreType.DMA((_NREG,))]
    )

    @functools.partial(
        pl.kernel,
        out_type=[
            jax.ShapeDtypeStruct((_NW * _NREG * _SEG,), jnp.int32),
            jax.ShapeDtypeStruct((_NW * 16,), jnp.int32),
        ],
        mesh=mesh,
        scratch_types=scratch,
        compiler_params=pltpu.CompilerParams(needs_layout_passes=False),
    )
    def part(pk_hbm, seg_hbm, cnt_hbm, in_v0, in_v1,
             b0, b1, b2, b3, b4, b5, b6, b7, cnt_v, in_sem, fl_sem):
        in_v = (in_v0, in_v1)
        rings = (b0, b1, b2, b3, b4, b5, b6, b7)
        c = lax.axis_index("c")
        s = lax.axis_index("s")
        w = s * _NSC + c
        lane = lax.iota(jnp.int32, 16)
        ebase = w * _EP

        rbase = w * (_EP // 1024)

        def start_in(g, b):
            pltpu.async_copy(pk_hbm.at[pl.ds(rbase + g * 4, 4), :],
                             in_v[b], in_sem.at[b])

        def wait_in(g, b):
            pltpu.make_async_copy(
                pk_hbm.at[pl.ds(rbase + g * 4, 4), :],
                in_v[b], in_sem.at[b]).wait()

        def flush_desc(rr, off, blkno):
            sbase = (w * _NREG + rr) * _SEG
            return pltpu.make_async_copy(
                rings[rr].at[pl.ds(pl.multiple_of(off, _BLK), _BLK)],
                seg_hbm.at[pl.ds(pl.multiple_of(sbase + blkno * _BLK, _BLK),
                                 _BLK)],
                fl_sem.at[rr])

        def group(b, row, i, bases):
            o = pl.multiple_of(i * 16, 16)
            v = in_v[b][row, pl.ds(o, 16)]
            rr_v = lax.shift_right_logical(v, _LOCBITS)
            local = lax.bitwise_and(v, (1 << _LOCBITS) - 1)
            new = []
            for rr in range(_NREG):
                m = rr_v == rr
                mi = m.astype(jnp.int32)
                scan = plsc.cumsum(mi)
                pos = lax.bitwise_and(bases[rr] + (scan - 1), _RING - 1)
                plsc.store_scatter(rings[rr], [pos], local, mask=m)
                tot = plsc.all_reduce_population_count(m)
                new.append(bases[rr] + tot)
            return tuple(new)

        def flush_two(bases, flushed, blks):
            nf, nb = [], []
            for rr in range(_NREG):
                fl = flushed[rr]
                bl = blks[rr]
                bs = jnp.max(bases[rr])
                for _ in range(2):
                    do = (bs - fl) >= _BLK

                    @pl.when(do & (bl > 0))
                    def _():
                        flush_desc(rr, 0, 0).wait()

                    @pl.when(do)
                    def _():
                        flush_desc(rr, lax.bitwise_and(fl, _RING - 1),
                                   bl).start()

                    fl = jnp.where(do, fl + _BLK, fl)
                    bl = jnp.where(do, bl + 1, bl)
                nf.append(fl)
                nb.append(bl)
            return tuple(nf), tuple(nb)

        def chunk(k2, state):
            bases, flushed, blks = state
            for b in range(2):
                g = 2 * k2 + b
                wait_in(g, b)

                for half in range(2):
                    for row in (2 * half, 2 * half + 1):
                        def gbody(i, bs, b=b, row=row):
                            return group(b, row, i, bs)

                        bases = lax.fori_loop(0, 1024 // 16, gbody, bases)
                    flushed, blks = flush_two(bases, flushed, blks)

                @pl.when(g + 2 < _PNCH)
                def _():
                    start_in(g + 2, b)
            return bases, flushed, blks

        start_in(0, 0)
        start_in(1, 1)
        zv = jnp.zeros((16,), jnp.int32)
        z = jnp.int32(0)
        state = ((zv,) * _NREG, (z,) * _NREG, (z,) * _NREG)
        bases, flushed, blks = lax.fori_loop(0, _PNCH // 2, chunk, state)

        cvec = jnp.zeros((16,), jnp.int32)
        for rr in range(_NREG):
            dump = _R + lax.bitwise_and(w * 64 + rr * 16 + lane, _DUMP - 1)
            base = jnp.max(bases[rr])

            def padb(i, carry):
                pos = lax.bitwise_and(base + i * 16 + lane, _RING - 1)
                plsc.store_scatter(rings[rr], [pos], dump)
                return carry

            lax.fori_loop(0, _BLK // 16, padb, 0)
            gap = lax.bitwise_and(-base, _BLK - 1)
            padded = base + gap
            fl = flushed[rr]
            bl = blks[rr]
            for _ in range(2):
                do = (padded - fl) >= _BLK

                @pl.when(do & (bl > 0))
                def _():
                    flush_desc(rr, 0, 0).wait()

                @pl.when(do)
                def _():
                    flush_desc(rr, lax.bitwise_and(fl, _RING - 1),
                               bl).start()

                fl = jnp.where(do, fl + _BLK, fl)
                bl = jnp.where(do, bl + 1, bl)

            @pl.when(bl > 0)
            def _():
                flush_desc(rr, 0, 0).wait()

            cvec = jnp.where(lane == rr, bl, cvec)

        cnt_v[...] = cvec
        pltpu.sync_copy(cnt_v, cnt_hbm.at[pl.ds(w * 16, 16)])

    return part(packed)


def _sc_scatter(seg_hbm, cnt_hbm, zeros_hbm):
    mesh = plsc.VectorSubcoreMesh(
        core_axis_name="c", subcore_axis_name="s",
        num_cores=_NSC, num_subcores=_NTILE)

    scratch = (
        [pltpu.VMEM((_BLK,), jnp.int32) for _ in range(4)]
        + [pltpu.VMEM((_BLK,), jnp.float32),
           pltpu.VMEM((16,), jnp.int32),
           pltpu.VMEM((16,), jnp.int32),
           pltpu.VMEM_SHARED((_RD,), jnp.float32)]
        + [pltpu.SemaphoreType.DMA((4,)),
           pltpu.SemaphoreType.DMA((4,))]
    )

    @functools.partial(
        pl.kernel,
        out_type=jax.ShapeDtypeStruct((_NBINS,), jnp.float32),
        mesh=mesh,
        scratch_types=scratch,
        compiler_params=pltpu.CompilerParams(needs_layout_passes=False),
    )
    def scat(seg, cnt, z_hbm, out_hbm, i0, i1, i2, i3, ones_v,
             ca_v, cb_v, bins_sh, in_sem, sc_sem):
        in_v = (i0, i1, i2, i3)
        c = lax.axis_index("c")
        s = lax.axis_index("s")
        lane = lax.iota(jnp.int32, 16)

        def fill_ones(i, carry):
            ones_v[pl.ds(pl.multiple_of(i * 16, 16), 16)] = (
                jnp.full((16,), 1.0, jnp.float32))
            return carry

        lax.fori_loop(0, _BLK // 16, fill_ones, 0)

        zoff = s * (_RD // _NTILE)
        woff = s * (_R // _NTILE)
        wa = 2 * s
        wb = 2 * s + 1

        def pass_body(k, carry):
            rr = _NSC * k + c
            lo = rr * _R
            pltpu.sync_copy(cnt.at[pl.ds(wa * 16, 16)], ca_v)
            pltpu.sync_copy(cnt.at[pl.ds(wb * 16, 16)], cb_v)
            na = jnp.max(jnp.where(lane == rr, ca_v[pl.ds(0, 16)], 0))
            nb = jnp.max(jnp.where(lane == rr, cb_v[pl.ds(0, 16)], 0))
            ntot = na + nb
            sba = (wa * _NREG + rr) * _SEG
            sbb = (wb * _NREG + rr) * _SEG

            def blk_src(i):
                off = jnp.where(i < na, sba + i * _BLK,
                                sbb + (i - na) * _BLK)
                return pl.ds(pl.multiple_of(off, 8), _BLK)

            def start_in(i, b):
                pltpu.async_copy(seg.at[blk_src(i)], in_v[b], in_sem.at[b])

            def wait_in(i, b):
                pltpu.make_async_copy(seg.at[blk_src(i)], in_v[b],
                                      in_sem.at[b]).wait()

            def start_scatter(b):
                pltpu.async_copy(ones_v, bins_sh.at[in_v[b]],
                                 sc_sem.at[b], add=True)

            def wait_scatter(b):
                pltpu.make_async_copy(ones_v, bins_sh.at[in_v[b]],
                                      sc_sem.at[b]).wait()

            pltpu.sync_copy(z_hbm.at[pl.ds(zoff, _RD // _NTILE)],
                            bins_sh.at[pl.ds(zoff, _RD // _NTILE)])

            @pl.when(ntot > 0)
            def _():
                start_in(0, 0)

            plsc.subcore_barrier()

            def consume4(k4, carry2):
                for bb in range(4):
                    i = 4 * k4 + bb
                    bn = (bb + 1) % 4

                    @pl.when(i < ntot)
                    def _():
                        wait_in(i, bb)

                        @pl.when(i >= 3)
                        def _():
                            wait_scatter(bn)

                        @pl.when(i + 1 < ntot)
                        def _():
                            start_in(i + 1, bn)

                        start_scatter(bb)
                return carry2

            lax.fori_loop(0, lax.div(ntot + 3, 4), consume4, 0)

            for bb in range(4):
                cond = jnp.bool_(False)
                for j in range(1, 4):
                    cond = cond | ((ntot >= j) &
                                   (lax.rem(ntot - j, 4) == bb))

                @pl.when(cond)
                def _():
                    wait_scatter(bb)

            plsc.subcore_barrier()
            pltpu.sync_copy(bins_sh.at[pl.ds(woff, _R // _NTILE)],
                            out_hbm.at[pl.ds(lo + woff, _R // _NTILE)])
            plsc.subcore_barrier()
            return carry

        lax.fori_loop(0, _NPASS, pass_body, 0)

    return scat(seg_hbm, cnt_hbm, zeros_hbm)


def _normalize(counts3):
    def body(c_ref, o_ref):
        v = c_ref[...]
        m = jnp.max(v)
        o_ref[...] = jnp.where(m > 0, v / m, v)

    return pl.pallas_call(
        body,
        grid=(_T,),
        in_specs=[pl.BlockSpec((1, _H, 2 * _W), lambda i: (i, 0, 0))],
        out_specs=pl.BlockSpec((1, _H, 2 * _W), lambda i: (i, 0, 0)),
        out_shape=jax.ShapeDtypeStruct(counts3.shape, jnp.float32),
    )(counts3)


def kernel(x, y, p, t):
    t2 = t.reshape(8192, 1024)
    mn, mx = _minmax(t2)
    packed = _packed_index(x.reshape(8192, 1024), y.reshape(8192, 1024),
                           p.reshape(8192, 1024), t2, mn, mx)
    seg, cnt = _sc_partition(packed)
    counts = _sc_scatter(seg, cnt, jnp.zeros((_RD,), jnp.float32))
    voxel = _normalize(counts.reshape(_T, _H, 2 * _W))
    return voxel.reshape(_T, _C, _H, _W)
